# baseline (speedup 1.0000x reference)
import functools

import jax
import jax.numpy as jnp
from jax import lax
from jax.experimental import pallas as pl
from jax.experimental.pallas import tpu as pltpu
from jax.experimental.pallas import tpu_sc as plsc


_NC = 2
_NS = 16
_NW = _NC * _NS
_LANES = 16


def _make_sc_pool(seq, batch, vocab_half, emb):
    cols = batch // _NW
    wide = 2 * emb
    half = _NS * cols
    nbuf = 4
    chunk = 40
    assert seq % chunk == 0 and chunk % nbuf == 0
    mesh = plsc.VectorSubcoreMesh(core_axis_name="c", subcore_axis_name="s")

    @functools.partial(
        pl.kernel,
        mesh=mesh,
        out_type=jax.ShapeDtypeStruct((batch, emb), jnp.float32),
        compiler_params=pltpu.CompilerParams(use_tc_tiling_on_sc=True),
        scratch_types=[
            pltpu.VMEM((chunk, cols), jnp.int32),
            pltpu.VMEM((nbuf, cols), jnp.int32),
            pltpu.VMEM((nbuf, cols), jnp.int32),
            pltpu.VMEM((nbuf, cols, wide), jnp.float32),
            pltpu.VMEM((cols, emb), jnp.float32),
            pltpu.VMEM_SHARED((2 * _NS * cols, wide), jnp.float32),
            pltpu.SemaphoreType.DMA((nbuf,)),
            pltpu.SemaphoreType.DMA((nbuf,)),
        ],
    )
    def sc_pool(x_hbm, emb_hbm, out_hbm, idx_v, gidx_v, dst_v, buf_v, res_v,
                acc_sh, gsem, ssem):
        cid = lax.axis_index("c")
        sid = lax.axis_index("s")
        wid = sid * _NC + cid
        base = wid * cols

        def fire(s, b):
            for i in range(cols // _LANES):
                sl = pl.ds(i * _LANES, _LANES)
                v = idx_v[s, sl]
                c = lax.iota(jnp.int32, _LANES) + (sid * cols + i * _LANES)
                dst_v[b, sl] = (v & 1) * half + c
                gidx_v[b, sl] = lax.shift_right_logical(v, 1)
            pltpu.async_copy(emb_hbm.at[gidx_v.at[b]], buf_v.at[b],
                             gsem.at[b])

        zeros = jnp.zeros((_LANES,), jnp.float32)

        def zrow(i, carry):
            for d in range(wide // _LANES):
                buf_v[0, i, pl.ds(d * _LANES, _LANES)] = zeros
            return carry

        lax.fori_loop(0, cols, zrow, 0)
        pltpu.sync_copy(buf_v.at[0], acc_sh.at[pl.ds(sid * cols, cols)])
        pltpu.sync_copy(buf_v.at[0],
                        acc_sh.at[pl.ds(half + sid * cols, cols)])

        def chunk_body(ci, carry):
            pltpu.sync_copy(
                x_hbm.at[pl.ds(ci * chunk, chunk), pl.ds(base, cols)], idx_v)
            for b in range(nbuf):
                fire(b, b)

            def group(g, carry2):
                for b in range(nbuf):
                    s = g * nbuf + b
                    pltpu.make_async_copy(
                        emb_hbm.at[gidx_v.at[b]], buf_v.at[b],
                        gsem.at[b]).wait()
                    pltpu.async_copy(buf_v.at[b], acc_sh.at[dst_v.at[b]],
                                     ssem.at[b], add=True)
                    pltpu.make_async_copy(
                        buf_v.at[b], acc_sh.at[dst_v.at[b]],
                        ssem.at[b]).wait()

                    @pl.when(s + nbuf < chunk)
                    def _():
                        fire(s + nbuf, b)
                return carry2

            lax.fori_loop(0, chunk // nbuf, group, 0)
            return carry

        lax.fori_loop(0, seq // chunk, chunk_body, 0)

        pltpu.sync_copy(acc_sh.at[pl.ds(sid * cols, cols)], buf_v.at[0])
        pltpu.sync_copy(acc_sh.at[pl.ds(half + sid * cols, cols)],
                        buf_v.at[1])

        def combine(i, carry):
            for d in range(emb // _LANES):
                sl = pl.ds(d * _LANES, _LANES)
                sh = pl.ds(emb + d * _LANES, _LANES)
                res_v[i, sl] = buf_v[0, i, sl] + buf_v[1, i, sh]
            return carry

        lax.fori_loop(0, cols, combine, 0)
        pltpu.sync_copy(res_v, out_hbm.at[pl.ds(base, cols)])

    return sc_pool


def _tc_head(sums, fc_w, fc_b2d, seq, blk):
    batch, emb = sums.shape
    out_dim = fc_w.shape[0]
    inv = 1.0 / seq

    def body(s_ref, w_ref, b_ref, o_ref):
        s = s_ref[...]
        w = w_ref[...]
        logits = lax.dot_general(
            s, w, (((1,), (1,)), ((), ())),
            preferred_element_type=jnp.float32,
        ) * inv + b_ref[...]
        m = jnp.max(logits, axis=-1, keepdims=True)
        e = jnp.exp(logits - m)
        lse = jnp.log(jnp.sum(e, axis=-1, keepdims=True)) + m
        o_ref[...] = logits - lse

    return pl.pallas_call(
        body,
        grid=(batch // blk,),
        in_specs=[
            pl.BlockSpec((blk, emb), lambda i: (i, 0)),
            pl.BlockSpec((out_dim, emb), lambda i: (0, 0)),
            pl.BlockSpec((1, out_dim), lambda i: (0, 0)),
        ],
        out_specs=pl.BlockSpec((blk, out_dim), lambda i: (i, 0)),
        out_shape=jax.ShapeDtypeStruct((batch, out_dim), jnp.float32),
    )(sums, fc_w, fc_b2d)


def kernel(x, embedding, fc_w, fc_b):
    seq, batch = x.shape
    vocab, emb = embedding.shape
    emb_pairs = embedding.reshape(vocab // 2, 2 * emb)
    sums = _make_sc_pool(seq, batch, vocab // 2, emb)(x, emb_pairs)
    return _tc_head(sums, fc_w, fc_b.reshape(1, -1), seq, blk=512)


# speedup vs baseline: 1.2001x; 1.2001x over previous
import functools

import jax
import jax.numpy as jnp
from jax import lax
from jax.experimental import pallas as pl
from jax.experimental.pallas import tpu as pltpu
from jax.experimental.pallas import tpu_sc as plsc


_NC = 2
_NS = 16
_NW = _NC * _NS
_LANES = 16


def _make_sc_pool(seq, batch, emb):
    cols = batch // _NW
    wide = 2 * emb
    nbuf = 4
    chunk = 40
    assert seq % chunk == 0 and chunk % nbuf == 0
    mesh = plsc.VectorSubcoreMesh(core_axis_name="c", subcore_axis_name="s")

    @functools.partial(
        pl.kernel,
        mesh=mesh,
        out_type=jax.ShapeDtypeStruct((batch, emb), jnp.float32),
        compiler_params=pltpu.CompilerParams(use_tc_tiling_on_sc=True),
        scratch_types=[
            pltpu.VMEM((chunk, cols), jnp.int32),
            pltpu.VMEM((nbuf, cols), jnp.int32),
            pltpu.VMEM((nbuf, cols), jnp.int32),
            pltpu.VMEM((nbuf, cols, wide), jnp.float32),
            pltpu.VMEM((cols, emb), jnp.float32),
            pltpu.VMEM_SHARED((_NS * cols, wide), jnp.float32),
            pltpu.SemaphoreType.DMA((nbuf,)),
            pltpu.SemaphoreType.DMA((nbuf,)),
        ],
    )
    def sc_pool(x_hbm, emb_hbm, out_hbm, idx_v, gidx_v, dst_v, buf_v, res_v,
                acc_sh, gsem, ssem):
        cid = lax.axis_index("c")
        sid = lax.axis_index("s")
        wid = sid * _NC + cid
        base = wid * cols

        def fire(s, b):
            for i in range(cols // _LANES):
                sl = pl.ds(i * _LANES, _LANES)
                gidx_v[b, sl] = idx_v[s, sl]
            pltpu.async_copy(emb_hbm.at[gidx_v.at[b]], buf_v.at[b],
                             gsem.at[b])

        zeros = jnp.zeros((_LANES,), jnp.float32)

        def zrow(i, carry):
            for d in range(wide // _LANES):
                buf_v[0, i, pl.ds(d * _LANES, _LANES)] = zeros
            return carry

        lax.fori_loop(0, cols, zrow, 0)
        pltpu.sync_copy(buf_v.at[0], acc_sh.at[pl.ds(sid * cols, cols)])

        for i in range(cols // _LANES):
            sl = pl.ds(i * _LANES, _LANES)
            c = lax.iota(jnp.int32, _LANES) + (sid * cols + i * _LANES)
            for b in range(nbuf):
                dst_v[b, sl] = c

        def chunk_body(ci, carry):
            pltpu.sync_copy(
                x_hbm.at[pl.ds(ci * chunk, chunk), pl.ds(base, cols)], idx_v)
            for b in range(nbuf):
                fire(b, b)

            def group(g, carry2):
                for b in range(nbuf):
                    s = g * nbuf + b
                    pltpu.make_async_copy(
                        emb_hbm.at[gidx_v.at[b]], buf_v.at[b],
                        gsem.at[b]).wait()
                    pltpu.async_copy(buf_v.at[b], acc_sh.at[dst_v.at[b]],
                                     ssem.at[b], add=True)
                    pltpu.make_async_copy(
                        buf_v.at[b], acc_sh.at[dst_v.at[b]],
                        ssem.at[b]).wait()

                    @pl.when(s + nbuf < chunk)
                    def _():
                        fire(s + nbuf, b)
                return carry2

            lax.fori_loop(0, chunk // nbuf, group, 0)
            return carry

        lax.fori_loop(0, seq // chunk, chunk_body, 0)

        pltpu.sync_copy(acc_sh.at[pl.ds(sid * cols, cols)], buf_v.at[0])

        def combine(i, carry):
            for d in range(emb // _LANES):
                sl = pl.ds(d * _LANES, _LANES)
                res_v[i, sl] = buf_v[0, i, sl]
            return carry

        lax.fori_loop(0, cols, combine, 0)
        pltpu.sync_copy(res_v, out_hbm.at[pl.ds(base, cols)])

    return sc_pool


def _tc_repack(emb_t, rows_blk):
    emb, vocab = emb_t.shape
    wide = 2 * emb

    def body(a_ref, o_ref):
        t = a_ref[...].T
        o_ref[...] = jnp.concatenate([t, jnp.zeros_like(t)], axis=1)

    return pl.pallas_call(
        body,
        grid=(pl.cdiv(vocab, rows_blk),),
        in_specs=[pl.BlockSpec((emb, rows_blk), lambda i: (0, i))],
        out_specs=pl.BlockSpec((rows_blk, wide), lambda i: (i, 0)),
        out_shape=jax.ShapeDtypeStruct((vocab, wide), jnp.float32),
    )(emb_t)


def _tc_head(sums, fc_w, fc_b2d, seq, blk):
    batch, emb = sums.shape
    out_dim = fc_w.shape[0]
    inv = 1.0 / seq

    def body(s_ref, w_ref, b_ref, o_ref):
        s = s_ref[...]
        w = w_ref[...]
        logits = lax.dot_general(
            s, w, (((1,), (1,)), ((), ())),
            preferred_element_type=jnp.float32,
        ) * inv + b_ref[...]
        m = jnp.max(logits, axis=-1, keepdims=True)
        e = jnp.exp(logits - m)
        lse = jnp.log(jnp.sum(e, axis=-1, keepdims=True)) + m
        o_ref[...] = logits - lse

    return pl.pallas_call(
        body,
        grid=(batch // blk,),
        in_specs=[
            pl.BlockSpec((blk, emb), lambda i: (i, 0)),
            pl.BlockSpec((out_dim, emb), lambda i: (0, 0)),
            pl.BlockSpec((1, out_dim), lambda i: (0, 0)),
        ],
        out_specs=pl.BlockSpec((blk, out_dim), lambda i: (i, 0)),
        out_shape=jax.ShapeDtypeStruct((batch, out_dim), jnp.float32),
    )(sums, fc_w, fc_b2d)


def kernel(x, embedding, fc_w, fc_b):
    seq, batch = x.shape
    vocab, emb = embedding.shape
    emb_wide = _tc_repack(embedding.T, rows_blk=2048)
    sums = _make_sc_pool(seq, batch, emb)(x, emb_wide)
    return _tc_head(sums, fc_w, fc_b.reshape(1, -1), seq, blk=512)


# speedup vs baseline: 1.8114x; 1.5093x over previous
import functools

import jax
import jax.numpy as jnp
from jax import lax
from jax.experimental import pallas as pl
from jax.experimental.pallas import tpu as pltpu
from jax.experimental.pallas import tpu_sc as plsc


_NC = 2
_NS = 16
_NW = _NC * _NS
_LANES = 16


def _make_sc_pool(seq, batch, emb, vblk, vshift):
    cols = batch // _NW
    nbuf = 4
    chunk = 40
    assert seq % chunk == 0 and chunk % nbuf == 0
    mesh = plsc.VectorSubcoreMesh(core_axis_name="c", subcore_axis_name="s")

    @functools.partial(
        pl.kernel,
        mesh=mesh,
        out_type=jax.ShapeDtypeStruct((batch, emb), jnp.float32),
        compiler_params=pltpu.CompilerParams(use_tc_tiling_on_sc=False),
        scratch_types=[
            pltpu.VMEM((chunk, cols), jnp.int32),
            pltpu.VMEM((nbuf, cols), jnp.int32),
            pltpu.VMEM((nbuf, cols), jnp.int32),
            pltpu.VMEM((nbuf, cols, emb), jnp.float32),
            pltpu.VMEM((cols, emb), jnp.float32),
            pltpu.VMEM_SHARED((_NS * cols, emb), jnp.float32),
            pltpu.SemaphoreType.DMA((nbuf,)),
            pltpu.SemaphoreType.DMA((nbuf,)),
        ],
    )
    def sc_pool(x_hbm, emb_hbm, out_hbm, idx_v, gidx_v, dst_v, buf_v, res_v,
                acc_sh, gsem, ssem):
        cid = lax.axis_index("c")
        sid = lax.axis_index("s")
        wid = sid * _NC + cid
        base = wid * cols

        def fire(s, b):
            for i in range(cols // _LANES):
                sl = pl.ds(i * _LANES, _LANES)
                v = idx_v[s, sl]
                gidx_v[b, sl] = (
                    (v & ~(vblk - 1))
                    | lax.shift_left(v & (vblk // 2 - 1), 1)
                    | (lax.shift_right_logical(v, vshift) & 1)
                )
            pltpu.async_copy(emb_hbm.at[gidx_v.at[b]], buf_v.at[b],
                             gsem.at[b])

        zeros = jnp.zeros((_LANES,), jnp.float32)

        def zrow(i, carry):
            for d in range(emb // _LANES):
                buf_v[0, i, pl.ds(d * _LANES, _LANES)] = zeros
            return carry

        lax.fori_loop(0, cols, zrow, 0)
        pltpu.sync_copy(buf_v.at[0], acc_sh.at[pl.ds(sid * cols, cols)])

        for i in range(cols // _LANES):
            sl = pl.ds(i * _LANES, _LANES)
            c = lax.iota(jnp.int32, _LANES) + (sid * cols + i * _LANES)
            for b in range(nbuf):
                dst_v[b, sl] = c

        def chunk_body(ci, carry):
            pltpu.sync_copy(
                x_hbm.at[pl.ds(ci * chunk, chunk), pl.ds(base, cols)], idx_v)
            for b in range(nbuf):
                fire(b, b)

            def group(g, carry2):
                for b in range(nbuf):
                    s = g * nbuf + b
                    pltpu.make_async_copy(
                        emb_hbm.at[gidx_v.at[b]], buf_v.at[b],
                        gsem.at[b]).wait()
                    pltpu.async_copy(buf_v.at[b], acc_sh.at[dst_v.at[b]],
                                     ssem.at[b], add=True)
                    pltpu.make_async_copy(
                        buf_v.at[b], acc_sh.at[dst_v.at[b]],
                        ssem.at[b]).wait()

                    @pl.when(s + nbuf < chunk)
                    def _():
                        fire(s + nbuf, b)
                return carry2

            lax.fori_loop(0, chunk // nbuf, group, 0)
            return carry

        lax.fori_loop(0, seq // chunk, chunk_body, 0)

        pltpu.sync_copy(acc_sh.at[pl.ds(sid * cols, cols)], buf_v.at[0])

        def combine(i, carry):
            for d in range(emb // _LANES):
                sl = pl.ds(d * _LANES, _LANES)
                res_v[i, sl] = buf_v[0, i, sl]
            return carry

        lax.fori_loop(0, cols, combine, 0)
        pltpu.sync_copy(res_v, out_hbm.at[pl.ds(base, cols)])

    return sc_pool


def _tc_repack(emb_t, v_blk):
    emb, vocab = emb_t.shape
    wide = 2 * emb
    hblk = v_blk // 2

    def body(a_ref, o_ref):
        t = a_ref[...].T
        o_ref[...] = jnp.concatenate([t[:hblk], t[hblk:]], axis=1)

    return pl.pallas_call(
        body,
        grid=(pl.cdiv(vocab, v_blk),),
        in_specs=[pl.BlockSpec((emb, v_blk), lambda i: (0, i))],
        out_specs=pl.BlockSpec((hblk, wide), lambda i: (i, 0)),
        out_shape=jax.ShapeDtypeStruct(
            (pl.cdiv(vocab, v_blk) * hblk, wide), jnp.float32),
    )(emb_t)


def _tc_head(sums, fc_w, fc_b2d, seq, blk):
    batch, emb = sums.shape
    out_dim = fc_w.shape[0]
    inv = 1.0 / seq

    def body(s_ref, w_ref, b_ref, o_ref):
        s = s_ref[...]
        w = w_ref[...]
        logits = lax.dot_general(
            s, w, (((1,), (1,)), ((), ())),
            preferred_element_type=jnp.float32,
        ) * inv + b_ref[...]
        m = jnp.max(logits, axis=-1, keepdims=True)
        e = jnp.exp(logits - m)
        lse = jnp.log(jnp.sum(e, axis=-1, keepdims=True)) + m
        o_ref[...] = logits - lse

    return pl.pallas_call(
        body,
        grid=(batch // blk,),
        in_specs=[
            pl.BlockSpec((blk, emb), lambda i: (i, 0)),
            pl.BlockSpec((out_dim, emb), lambda i: (0, 0)),
            pl.BlockSpec((1, out_dim), lambda i: (0, 0)),
        ],
        out_specs=pl.BlockSpec((blk, out_dim), lambda i: (i, 0)),
        out_shape=jax.ShapeDtypeStruct((batch, out_dim), jnp.float32),
    )(sums, fc_w, fc_b2d)


def kernel(x, embedding, fc_w, fc_b):
    seq, batch = x.shape
    vocab, emb = embedding.shape
    vblk = 4096
    emb_dense = _tc_repack(embedding.T, v_blk=vblk).reshape(-1, emb)
    sums = _make_sc_pool(seq, batch, emb, vblk, 11)(x, emb_dense)
    return _tc_head(sums, fc_w, fc_b.reshape(1, -1), seq, blk=512)


# speedup vs baseline: 2.1266x; 1.1740x over previous
import functools

import jax
import jax.numpy as jnp
from jax import lax
from jax.experimental import pallas as pl
from jax.experimental.pallas import tpu as pltpu
from jax.experimental.pallas import tpu_sc as plsc


_NC = 2
_NS = 16
_NW = _NC * _NS
_LANES = 16


def _make_sc_pool(seq, batch, emb, vblk, vshift):
    cols = batch // _NW
    nbuf = 4
    chunk = 40
    assert seq % chunk == 0 and chunk % nbuf == 0
    mesh = plsc.VectorSubcoreMesh(core_axis_name="c", subcore_axis_name="s")

    @functools.partial(
        pl.kernel,
        mesh=mesh,
        out_type=jax.ShapeDtypeStruct((batch, emb), jnp.float32),
        compiler_params=pltpu.CompilerParams(use_tc_tiling_on_sc=False),
        scratch_types=[
            pltpu.VMEM((chunk, cols), jnp.int32),
            pltpu.VMEM((nbuf, cols), jnp.int32),
            pltpu.VMEM((nbuf, cols), jnp.int32),
            pltpu.VMEM((nbuf, cols, emb), jnp.float32),
            pltpu.VMEM((cols, emb), jnp.float32),
            pltpu.VMEM_SHARED((_NS * cols, emb), jnp.float32),
            pltpu.SemaphoreType.DMA((nbuf,)),
            pltpu.SemaphoreType.DMA((nbuf,)),
        ],
    )
    def sc_pool(x_hbm, emb_hbm, out_hbm, idx_v, gidx_v, dst_v, buf_v, res_v,
                acc_sh, gsem, ssem):
        cid = lax.axis_index("c")
        sid = lax.axis_index("s")
        wid = sid * _NC + cid
        base = wid * cols

        def fire(s, b):
            for i in range(cols // _LANES):
                sl = pl.ds(i * _LANES, _LANES)
                v = idx_v[s, sl]
                gidx_v[b, sl] = (
                    (v & ~(vblk - 1))
                    | lax.shift_left(v & (vblk // 2 - 1), 1)
                    | (lax.shift_right_logical(v, vshift) & 1)
                )
            pltpu.async_copy(emb_hbm.at[gidx_v.at[b]], buf_v.at[b],
                             gsem.at[b])

        zeros = jnp.zeros((_LANES,), jnp.float32)

        def zrow(i, carry):
            for d in range(emb // _LANES):
                buf_v[0, i, pl.ds(d * _LANES, _LANES)] = zeros
            return carry

        lax.fori_loop(0, cols, zrow, 0)
        pltpu.sync_copy(buf_v.at[0], acc_sh.at[pl.ds(sid * cols, cols)])

        for i in range(cols // _LANES):
            sl = pl.ds(i * _LANES, _LANES)
            c = lax.iota(jnp.int32, _LANES) + (sid * cols + i * _LANES)
            for b in range(nbuf):
                dst_v[b, sl] = c

        def chunk_body(ci, carry):
            pltpu.sync_copy(
                x_hbm.at[pl.ds(ci * chunk, chunk), pl.ds(base, cols)], idx_v)
            for b in range(nbuf):
                fire(b, b)

            def group(g, carry2):
                for b in range(nbuf):
                    s = g * nbuf + b
                    pltpu.make_async_copy(
                        emb_hbm.at[gidx_v.at[b]], buf_v.at[b],
                        gsem.at[b]).wait()
                    pltpu.async_copy(buf_v.at[b], acc_sh.at[dst_v.at[b]],
                                     ssem.at[b], add=True)
                    pltpu.make_async_copy(
                        buf_v.at[b], acc_sh.at[dst_v.at[b]],
                        ssem.at[b]).wait()

                    @pl.when(s + nbuf < chunk)
                    def _():
                        fire(s + nbuf, b)
                return carry2

            lax.fori_loop(0, chunk // nbuf, group, 0)
            return carry

        lax.fori_loop(0, seq // chunk, chunk_body, 0)

        pltpu.sync_copy(acc_sh.at[pl.ds(sid * cols, cols)], buf_v.at[0])

        def combine(i, carry):
            for d in range(emb // _LANES):
                sl = pl.ds(d * _LANES, _LANES)
                res_v[i, sl] = buf_v[0, i, sl]
            return carry

        lax.fori_loop(0, cols, combine, 0)
        pltpu.sync_copy(res_v, out_hbm.at[pl.ds(base, cols)])

    return sc_pool


def _tc_repack(emb_t, v_blk):
    emb, vocab = emb_t.shape
    wide = 2 * emb
    hblk = v_blk // 2

    def body(a_ref, o_ref):
        t = a_ref[...].T
        o_ref[...] = jnp.concatenate([t[:hblk], t[hblk:]], axis=1)

    return pl.pallas_call(
        body,
        grid=(pl.cdiv(vocab, v_blk),),
        in_specs=[pl.BlockSpec((emb, v_blk), lambda i: (0, i))],
        out_specs=pl.BlockSpec((hblk, wide), lambda i: (i, 0)),
        out_shape=jax.ShapeDtypeStruct(
            (pl.cdiv(vocab, v_blk) * hblk, wide), jnp.float32),
    )(emb_t)


def _tc_head(sums, fc_w, fc_b2d, seq, blk):
    batch, emb = sums.shape
    out_dim = fc_w.shape[0]
    inv = 1.0 / seq

    def body(s_ref, w_ref, b_ref, o_ref):
        s = s_ref[...]
        w = w_ref[...]
        logits = lax.dot_general(
            s, w, (((1,), (1,)), ((), ())),
            preferred_element_type=jnp.float32,
        ) * inv + b_ref[...]
        m = jnp.max(logits, axis=-1, keepdims=True)
        e = jnp.exp(logits - m)
        lse = jnp.log(jnp.sum(e, axis=-1, keepdims=True)) + m
        o_ref[...] = logits - lse

    return pl.pallas_call(
        body,
        grid=(batch // blk,),
        in_specs=[
            pl.BlockSpec((blk, emb), lambda i: (i, 0)),
            pl.BlockSpec((out_dim, emb), lambda i: (0, 0)),
            pl.BlockSpec((1, out_dim), lambda i: (0, 0)),
        ],
        out_specs=pl.BlockSpec((blk, out_dim), lambda i: (i, 0)),
        out_shape=jax.ShapeDtypeStruct((batch, out_dim), jnp.float32),
    )(sums, fc_w, fc_b2d)


def kernel(x, embedding, fc_w, fc_b):
    seq, batch = x.shape
    vocab, emb = embedding.shape
    vblk = 8192
    emb_dense = _tc_repack(embedding.T, v_blk=vblk).reshape(-1, emb)
    sums = _make_sc_pool(seq, batch, emb, vblk, 12)(x, emb_dense)
    return _tc_head(sums, fc_w, fc_b.reshape(1, -1), seq, blk=512)


# speedup vs baseline: 2.3155x; 1.0888x over previous
import functools

import jax
import jax.numpy as jnp
from jax import lax
from jax.experimental import pallas as pl
from jax.experimental.pallas import tpu as pltpu
from jax.experimental.pallas import tpu_sc as plsc


_NC = 2
_NS = 16
_NW = _NC * _NS
_LANES = 16


def _make_sc_pool(seq, batch, emb, vblk, vshift):
    cols = batch // _NW
    nbuf = 4
    chunk = 40
    assert seq % chunk == 0 and chunk % nbuf == 0
    mesh = plsc.VectorSubcoreMesh(core_axis_name="c", subcore_axis_name="s")

    @functools.partial(
        pl.kernel,
        mesh=mesh,
        out_type=jax.ShapeDtypeStruct((batch, emb), jnp.float32),
        compiler_params=pltpu.CompilerParams(use_tc_tiling_on_sc=False),
        scratch_types=[
            pltpu.VMEM((chunk, cols), jnp.int32),
            pltpu.VMEM((nbuf, cols), jnp.int32),
            pltpu.VMEM((nbuf, cols), jnp.int32),
            pltpu.VMEM((nbuf, cols, emb), jnp.float32),
            pltpu.VMEM((cols, emb), jnp.float32),
            pltpu.VMEM_SHARED((_NS * cols, emb), jnp.float32),
            pltpu.SemaphoreType.DMA((nbuf,)),
            pltpu.SemaphoreType.DMA((nbuf,)),
        ],
    )
    def sc_pool(x_hbm, emb_hbm, out_hbm, idx_v, gidx_v, dst_v, buf_v, res_v,
                acc_sh, gsem, ssem):
        cid = lax.axis_index("c")
        sid = lax.axis_index("s")
        wid = sid * _NC + cid
        base = wid * cols

        def fire(s, b):
            for i in range(cols // _LANES):
                sl = pl.ds(i * _LANES, _LANES)
                v = idx_v[s, sl]
                gidx_v[b, sl] = (
                    (v & ~(vblk - 1))
                    | lax.shift_left(v & (vblk // 2 - 1), 1)
                    | (lax.shift_right_logical(v, vshift) & 1)
                )
            pltpu.async_copy(emb_hbm.at[gidx_v.at[b]], buf_v.at[b],
                             gsem.at[b])

        zeros = jnp.zeros((_LANES,), jnp.float32)

        def zrow(i, carry):
            for d in range(emb // _LANES):
                buf_v[0, i, pl.ds(d * _LANES, _LANES)] = zeros
            return carry

        lax.fori_loop(0, cols, zrow, 0)
        pltpu.sync_copy(buf_v.at[0], acc_sh.at[pl.ds(sid * cols, cols)])

        for i in range(cols // _LANES):
            sl = pl.ds(i * _LANES, _LANES)
            c = lax.iota(jnp.int32, _LANES) + (sid * cols + i * _LANES)
            for b in range(nbuf):
                dst_v[b, sl] = c

        def chunk_body(ci, carry):
            pltpu.sync_copy(
                x_hbm.at[pl.ds(ci * chunk, chunk), pl.ds(base, cols)], idx_v)
            for b in range(nbuf):
                fire(b, b)

            def group(g, carry2):
                for b in range(nbuf):
                    s = g * nbuf + b
                    pltpu.make_async_copy(
                        emb_hbm.at[gidx_v.at[b]], buf_v.at[b],
                        gsem.at[b]).wait()
                    pltpu.async_copy(buf_v.at[b], acc_sh.at[dst_v.at[b]],
                                     ssem.at[b], add=True)
                    pltpu.make_async_copy(
                        buf_v.at[b], acc_sh.at[dst_v.at[b]],
                        ssem.at[b]).wait()

                    @pl.when(s + nbuf < chunk)
                    def _():
                        fire(s + nbuf, b)
                return carry2

            lax.fori_loop(0, chunk // nbuf, group, 0)
            return carry

        lax.fori_loop(0, seq // chunk, chunk_body, 0)

        pltpu.sync_copy(acc_sh.at[pl.ds(sid * cols, cols)], buf_v.at[0])

        def combine(i, carry):
            for d in range(emb // _LANES):
                sl = pl.ds(d * _LANES, _LANES)
                res_v[i, sl] = buf_v[0, i, sl]
            return carry

        lax.fori_loop(0, cols, combine, 0)
        pltpu.sync_copy(res_v, out_hbm.at[pl.ds(base, cols)])

    return sc_pool


def _tc_repack(emb_t, v_blk):
    emb, vocab = emb_t.shape
    wide = 2 * emb
    hblk = v_blk // 2

    def body(a_ref, o_ref):
        t = a_ref[...].T
        o_ref[...] = jnp.concatenate([t[:hblk], t[hblk:]], axis=1)

    return pl.pallas_call(
        body,
        grid=(pl.cdiv(vocab, v_blk),),
        in_specs=[pl.BlockSpec((emb, v_blk), lambda i: (0, i))],
        out_specs=pl.BlockSpec((hblk, wide), lambda i: (i, 0)),
        out_shape=jax.ShapeDtypeStruct(
            (pl.cdiv(vocab, v_blk) * hblk, wide), jnp.float32),
    )(emb_t)


def _tc_head(sums, fc_w, fc_b2d, seq, blk):
    batch, emb = sums.shape
    out_dim = fc_w.shape[0]
    inv = 1.0 / seq

    def body(s_ref, w_ref, b_ref, o_ref):
        s = s_ref[...]
        w = w_ref[...]
        logits = lax.dot_general(
            s, w, (((1,), (1,)), ((), ())),
            preferred_element_type=jnp.float32,
        ) * inv + b_ref[...]
        m = jnp.max(logits, axis=-1, keepdims=True)
        e = jnp.exp(logits - m)
        lse = jnp.log(jnp.sum(e, axis=-1, keepdims=True)) + m
        o_ref[...] = logits - lse

    return pl.pallas_call(
        body,
        grid=(batch // blk,),
        in_specs=[
            pl.BlockSpec((blk, emb), lambda i: (i, 0)),
            pl.BlockSpec((out_dim, emb), lambda i: (0, 0)),
            pl.BlockSpec((1, out_dim), lambda i: (0, 0)),
        ],
        out_specs=pl.BlockSpec((blk, out_dim), lambda i: (i, 0)),
        out_shape=jax.ShapeDtypeStruct((batch, out_dim), jnp.float32),
    )(sums, fc_w, fc_b2d)


def kernel(x, embedding, fc_w, fc_b):
    seq, batch = x.shape
    vocab, emb = embedding.shape
    vblk = 16384
    emb_dense = _tc_repack(embedding.T, v_blk=vblk).reshape(-1, emb)
    sums = _make_sc_pool(seq, batch, emb, vblk, 13)(x, emb_dense)
    return _tc_head(sums, fc_w, fc_b.reshape(1, -1), seq, blk=512)


# speedup vs baseline: 2.4082x; 1.0400x over previous
import functools

import jax
import jax.numpy as jnp
from jax import lax
from jax.experimental import pallas as pl
from jax.experimental.pallas import tpu as pltpu
from jax.experimental.pallas import tpu_sc as plsc


_NC = 2
_NS = 16
_NW = _NC * _NS
_LANES = 16


def _make_sc_pool(seq, batch, emb, vblk, vshift):
    cols = batch // _NW
    nbuf = 4
    chunk = 40
    assert seq % chunk == 0 and chunk % nbuf == 0
    mesh = plsc.VectorSubcoreMesh(core_axis_name="c", subcore_axis_name="s")

    @functools.partial(
        pl.kernel,
        mesh=mesh,
        out_type=jax.ShapeDtypeStruct((batch, emb), jnp.float32),
        compiler_params=pltpu.CompilerParams(use_tc_tiling_on_sc=False),
        scratch_types=[
            pltpu.VMEM((chunk, cols), jnp.int32),
            pltpu.VMEM((nbuf, cols), jnp.int32),
            pltpu.VMEM((nbuf, cols), jnp.int32),
            pltpu.VMEM((nbuf, cols, emb), jnp.float32),
            pltpu.VMEM((cols, emb), jnp.float32),
            pltpu.VMEM_SHARED((_NS * cols, emb), jnp.float32),
            pltpu.SemaphoreType.DMA((nbuf,)),
            pltpu.SemaphoreType.DMA((nbuf,)),
        ],
    )
    def sc_pool(x_hbm, emb_hbm, out_hbm, idx_v, gidx_v, dst_v, buf_v, res_v,
                acc_sh, gsem, ssem):
        cid = lax.axis_index("c")
        sid = lax.axis_index("s")
        wid = sid * _NC + cid
        base = wid * cols

        def fire(s, b):
            for i in range(cols // _LANES):
                sl = pl.ds(i * _LANES, _LANES)
                v = idx_v[s, sl]
                gidx_v[b, sl] = (
                    (v & ~(vblk - 1))
                    | lax.shift_left(v & (vblk // 2 - 1), 1)
                    | (lax.shift_right_logical(v, vshift) & 1)
                )
            pltpu.async_copy(emb_hbm.at[gidx_v.at[b]], buf_v.at[b],
                             gsem.at[b])

        zeros = jnp.zeros((_LANES,), jnp.float32)

        def zrow(i, carry):
            for d in range(emb // _LANES):
                buf_v[0, i, pl.ds(d * _LANES, _LANES)] = zeros
            return carry

        lax.fori_loop(0, cols, zrow, 0)
        pltpu.sync_copy(buf_v.at[0], acc_sh.at[pl.ds(sid * cols, cols)])

        for i in range(cols // _LANES):
            sl = pl.ds(i * _LANES, _LANES)
            c = lax.iota(jnp.int32, _LANES) + (sid * cols + i * _LANES)
            for b in range(nbuf):
                dst_v[b, sl] = c

        def chunk_body(ci, carry):
            pltpu.sync_copy(
                x_hbm.at[pl.ds(ci * chunk, chunk), pl.ds(base, cols)], idx_v)
            for b in range(nbuf):
                fire(b, b)

            def group(g, carry2):
                for b in range(nbuf):
                    s = g * nbuf + b
                    pltpu.make_async_copy(
                        emb_hbm.at[gidx_v.at[b]], buf_v.at[b],
                        gsem.at[b]).wait()
                    pltpu.async_copy(buf_v.at[b], acc_sh.at[dst_v.at[b]],
                                     ssem.at[b], add=True)
                    pltpu.make_async_copy(
                        buf_v.at[b], acc_sh.at[dst_v.at[b]],
                        ssem.at[b]).wait()

                    @pl.when(s + nbuf < chunk)
                    def _():
                        fire(s + nbuf, b)
                return carry2

            lax.fori_loop(0, chunk // nbuf, group, 0)
            return carry

        lax.fori_loop(0, seq // chunk, chunk_body, 0)

        pltpu.sync_copy(acc_sh.at[pl.ds(sid * cols, cols)], buf_v.at[0])

        def combine(i, carry):
            for d in range(emb // _LANES):
                sl = pl.ds(d * _LANES, _LANES)
                res_v[i, sl] = buf_v[0, i, sl]
            return carry

        lax.fori_loop(0, cols, combine, 0)
        pltpu.sync_copy(res_v, out_hbm.at[pl.ds(base, cols)])

    return sc_pool


def _tc_repack(emb_t, v_blk):
    emb, vocab = emb_t.shape
    wide = 2 * emb
    hblk = v_blk // 2

    def body(a_ref, o_ref):
        t = a_ref[...].T
        o_ref[...] = jnp.concatenate([t[:hblk], t[hblk:]], axis=1)

    return pl.pallas_call(
        body,
        grid=(pl.cdiv(vocab, v_blk),),
        in_specs=[pl.BlockSpec((emb, v_blk), lambda i: (0, i))],
        out_specs=pl.BlockSpec((hblk, wide), lambda i: (i, 0)),
        out_shape=jax.ShapeDtypeStruct(
            (pl.cdiv(vocab, v_blk) * hblk, wide), jnp.float32),
    )(emb_t)


def _tc_head(sums, fc_w, fc_b2d, seq, blk):
    batch, emb = sums.shape
    out_dim = fc_w.shape[0]
    inv = 1.0 / seq

    def body(s_ref, w_ref, b_ref, o_ref):
        s = s_ref[...]
        w = w_ref[...]
        logits = lax.dot_general(
            s, w, (((1,), (1,)), ((), ())),
            preferred_element_type=jnp.float32,
        ) * inv + b_ref[...]
        m = jnp.max(logits, axis=-1, keepdims=True)
        e = jnp.exp(logits - m)
        lse = jnp.log(jnp.sum(e, axis=-1, keepdims=True)) + m
        o_ref[...] = logits - lse

    return pl.pallas_call(
        body,
        grid=(batch // blk,),
        in_specs=[
            pl.BlockSpec((blk, emb), lambda i: (i, 0)),
            pl.BlockSpec((out_dim, emb), lambda i: (0, 0)),
            pl.BlockSpec((1, out_dim), lambda i: (0, 0)),
        ],
        out_specs=pl.BlockSpec((blk, out_dim), lambda i: (i, 0)),
        out_shape=jax.ShapeDtypeStruct((batch, out_dim), jnp.float32),
    )(sums, fc_w, fc_b2d)


def kernel(x, embedding, fc_w, fc_b):
    seq, batch = x.shape
    vocab, emb = embedding.shape
    vblk = 32768
    emb_dense = _tc_repack(embedding.T, v_blk=vblk).reshape(-1, emb)
    sums = _make_sc_pool(seq, batch, emb, vblk, 14)(x, emb_dense)
    return _tc_head(sums, fc_w, fc_b.reshape(1, -1), seq, blk=512)


# speedup vs baseline: 2.4111x; 1.0012x over previous
import functools

import jax
import jax.numpy as jnp
from jax import lax
from jax.experimental import pallas as pl
from jax.experimental.pallas import tpu as pltpu
from jax.experimental.pallas import tpu_sc as plsc


_NC = 2
_NS = 16
_NW = _NC * _NS
_LANES = 16


def _make_sc_pool(seq, batch, emb, vblk, vshift):
    cols = batch // _NW
    nbuf = 4
    chunk = 40
    assert seq % chunk == 0 and chunk % nbuf == 0
    mesh = plsc.VectorSubcoreMesh(core_axis_name="c", subcore_axis_name="s")

    @functools.partial(
        pl.kernel,
        mesh=mesh,
        out_type=jax.ShapeDtypeStruct((batch, emb), jnp.float32),
        compiler_params=pltpu.CompilerParams(use_tc_tiling_on_sc=False),
        scratch_types=[
            pltpu.VMEM((chunk, cols), jnp.int32),
            pltpu.VMEM((nbuf, cols), jnp.int32),
            pltpu.VMEM((nbuf, cols), jnp.int32),
            pltpu.VMEM((nbuf, cols, emb), jnp.float32),
            pltpu.VMEM((cols, emb), jnp.float32),
            pltpu.VMEM_SHARED((_NS * cols, emb), jnp.float32),
            pltpu.SemaphoreType.DMA((nbuf,)),
            pltpu.SemaphoreType.DMA((nbuf,)),
        ],
    )
    def sc_pool(x_hbm, emb_hbm, out_hbm, idx_v, gidx_v, dst_v, buf_v, res_v,
                acc_sh, gsem, ssem):
        cid = lax.axis_index("c")
        sid = lax.axis_index("s")
        wid = sid * _NC + cid
        base = wid * cols

        def fire(s, b):
            for i in range(cols // _LANES):
                sl = pl.ds(i * _LANES, _LANES)
                v = idx_v[s, sl]
                gidx_v[b, sl] = (
                    (v & ~(vblk - 1))
                    | lax.shift_left(v & (vblk // 2 - 1), 1)
                    | (lax.shift_right_logical(v, vshift) & 1)
                )
            pltpu.async_copy(emb_hbm.at[gidx_v.at[b]], buf_v.at[b],
                             gsem.at[b])

        zeros = jnp.zeros((_LANES,), jnp.float32)

        def zrow(i, carry):
            for d in range(emb // _LANES):
                buf_v[0, i, pl.ds(d * _LANES, _LANES)] = zeros
            return carry

        lax.fori_loop(0, cols, zrow, 0)
        pltpu.sync_copy(buf_v.at[0], acc_sh.at[pl.ds(sid * cols, cols)])

        for i in range(cols // _LANES):
            sl = pl.ds(i * _LANES, _LANES)
            c = lax.iota(jnp.int32, _LANES) + (sid * cols + i * _LANES)
            for b in range(nbuf):
                dst_v[b, sl] = c

        def chunk_body(ci, carry):
            pltpu.sync_copy(
                x_hbm.at[pl.ds(ci * chunk, chunk), pl.ds(base, cols)], idx_v)
            for b in range(nbuf):
                fire(b, b)

            def group(g, carry2):
                for b in range(nbuf):
                    s = g * nbuf + b
                    pltpu.make_async_copy(
                        emb_hbm.at[gidx_v.at[b]], buf_v.at[b],
                        gsem.at[b]).wait()
                    pltpu.async_copy(buf_v.at[b], acc_sh.at[dst_v.at[b]],
                                     ssem.at[b], add=True)
                    pltpu.make_async_copy(
                        buf_v.at[b], acc_sh.at[dst_v.at[b]],
                        ssem.at[b]).wait()

                    @pl.when(s + nbuf < chunk)
                    def _():
                        fire(s + nbuf, b)
                return carry2

            lax.fori_loop(0, chunk // nbuf, group, 0)
            return carry

        lax.fori_loop(0, seq // chunk, chunk_body, 0)

        pltpu.sync_copy(acc_sh.at[pl.ds(sid * cols, cols)], buf_v.at[0])

        def combine(i, carry):
            for d in range(emb // _LANES):
                sl = pl.ds(d * _LANES, _LANES)
                res_v[i, sl] = buf_v[0, i, sl]
            return carry

        lax.fori_loop(0, cols, combine, 0)
        pltpu.sync_copy(res_v, out_hbm.at[pl.ds(base, cols)])

    return sc_pool


def _tc_repack(emb_t, v_blk):
    emb, vocab = emb_t.shape
    wide = 2 * emb
    hblk = v_blk // 2

    def body(a_ref, o_ref):
        a = a_ref[...]
        iden = (lax.broadcasted_iota(jnp.int32, (emb, emb), 0)
                == lax.broadcasted_iota(jnp.int32, (emb, emb), 1)
                ).astype(jnp.float32)
        t = lax.dot_general(a, iden, (((0,), (0,)), ((), ())),
                            preferred_element_type=jnp.float32)
        o_ref[...] = jnp.concatenate([t[:hblk], t[hblk:]], axis=1)

    return pl.pallas_call(
        body,
        grid=(pl.cdiv(vocab, v_blk),),
        in_specs=[pl.BlockSpec((emb, v_blk), lambda i: (0, i))],
        out_specs=pl.BlockSpec((hblk, wide), lambda i: (i, 0)),
        out_shape=jax.ShapeDtypeStruct(
            (pl.cdiv(vocab, v_blk) * hblk, wide), jnp.float32),
    )(emb_t)


def _tc_head(sums, fc_w, fc_b2d, seq, blk):
    batch, emb = sums.shape
    out_dim = fc_w.shape[0]
    inv = 1.0 / seq

    def body(s_ref, w_ref, b_ref, o_ref):
        s = s_ref[...]
        w = w_ref[...]
        logits = lax.dot_general(
            s, w, (((1,), (1,)), ((), ())),
            preferred_element_type=jnp.float32,
        ) * inv + b_ref[...]
        m = jnp.max(logits, axis=-1, keepdims=True)
        e = jnp.exp(logits - m)
        lse = jnp.log(jnp.sum(e, axis=-1, keepdims=True)) + m
        o_ref[...] = logits - lse

    return pl.pallas_call(
        body,
        grid=(batch // blk,),
        in_specs=[
            pl.BlockSpec((blk, emb), lambda i: (i, 0)),
            pl.BlockSpec((out_dim, emb), lambda i: (0, 0)),
            pl.BlockSpec((1, out_dim), lambda i: (0, 0)),
        ],
        out_specs=pl.BlockSpec((blk, out_dim), lambda i: (i, 0)),
        out_shape=jax.ShapeDtypeStruct((batch, out_dim), jnp.float32),
    )(sums, fc_w, fc_b2d)


def kernel(x, embedding, fc_w, fc_b):
    seq, batch = x.shape
    vocab, emb = embedding.shape
    vblk = 32768
    emb_dense = _tc_repack(embedding.T, v_blk=vblk).reshape(-1, emb)
    sums = _make_sc_pool(seq, batch, emb, vblk, 14)(x, emb_dense)
    return _tc_head(sums, fc_w, fc_b.reshape(1, -1), seq, blk=512)


# speedup vs baseline: 2.4152x; 1.0017x over previous
import functools

import jax
import jax.numpy as jnp
from jax import lax
from jax.experimental import pallas as pl
from jax.experimental.pallas import tpu as pltpu
from jax.experimental.pallas import tpu_sc as plsc


_NC = 2
_NS = 16
_NW = _NC * _NS
_LANES = 16


def _make_sc_pool(seq, batch, emb, vblk, vshift):
    cols = batch // _NW
    nbuf = 4
    chunk = 40
    assert seq % chunk == 0 and chunk % nbuf == 0
    mesh = plsc.VectorSubcoreMesh(core_axis_name="c", subcore_axis_name="s")

    @functools.partial(
        pl.kernel,
        mesh=mesh,
        out_type=jax.ShapeDtypeStruct((batch, emb), jnp.float32),
        compiler_params=pltpu.CompilerParams(use_tc_tiling_on_sc=False),
        scratch_types=[
            pltpu.VMEM((chunk, cols), jnp.int32),
            pltpu.VMEM((nbuf, cols), jnp.int32),
            pltpu.VMEM((nbuf, cols), jnp.int32),
            pltpu.VMEM((nbuf, cols, emb), jnp.float32),
            pltpu.VMEM((cols, emb), jnp.float32),
            pltpu.VMEM_SHARED((_NS * cols, emb), jnp.float32),
            pltpu.SemaphoreType.DMA((nbuf,)),
            pltpu.SemaphoreType.DMA((nbuf,)),
        ],
    )
    def sc_pool(x_hbm, emb_hbm, out_hbm, idx_v, gidx_v, dst_v, buf_v, res_v,
                acc_sh, gsem, ssem):
        cid = lax.axis_index("c")
        sid = lax.axis_index("s")
        wid = sid * _NC + cid
        base = wid * cols

        def fire(s, b):
            for i in range(cols // _LANES):
                sl = pl.ds(i * _LANES, _LANES)
                v = idx_v[s, sl]
                gidx_v[b, sl] = (
                    (v & ~(vblk - 1))
                    | lax.shift_left(v & (vblk // 2 - 1), 1)
                    | (lax.shift_right_logical(v, vshift) & 1)
                )
            pltpu.async_copy(emb_hbm.at[gidx_v.at[b]], buf_v.at[b],
                             gsem.at[b])

        zeros = jnp.zeros((_LANES,), jnp.float32)

        def zrow(i, carry):
            for d in range(emb // _LANES):
                buf_v[0, i, pl.ds(d * _LANES, _LANES)] = zeros
            return carry

        lax.fori_loop(0, cols, zrow, 0)
        pltpu.sync_copy(buf_v.at[0], acc_sh.at[pl.ds(sid * cols, cols)])

        for i in range(cols // _LANES):
            sl = pl.ds(i * _LANES, _LANES)
            c = lax.iota(jnp.int32, _LANES) + (sid * cols + i * _LANES)
            for b in range(nbuf):
                dst_v[b, sl] = c

        def chunk_body(ci, carry):
            pltpu.sync_copy(
                x_hbm.at[pl.ds(ci * chunk, chunk), pl.ds(base, cols)], idx_v)
            for b in range(nbuf):
                fire(b, b)

            def group(g, carry2):
                for b in range(nbuf):
                    s = g * nbuf + b
                    pltpu.make_async_copy(
                        emb_hbm.at[gidx_v.at[b]], buf_v.at[b],
                        gsem.at[b]).wait()
                    pltpu.async_copy(buf_v.at[b], acc_sh.at[dst_v.at[b]],
                                     ssem.at[b], add=True)
                    pltpu.make_async_copy(
                        buf_v.at[b], acc_sh.at[dst_v.at[b]],
                        ssem.at[b]).wait()

                    @pl.when(s + nbuf < chunk)
                    def _():
                        fire(s + nbuf, b)
                return carry2

            lax.fori_loop(0, chunk // nbuf, group, 0)
            return carry

        lax.fori_loop(0, seq // chunk, chunk_body, 0)

        pltpu.sync_copy(acc_sh.at[pl.ds(sid * cols, cols)], buf_v.at[0])

        def combine(i, carry):
            for d in range(emb // _LANES):
                sl = pl.ds(d * _LANES, _LANES)
                res_v[i, sl] = buf_v[0, i, sl]
            return carry

        lax.fori_loop(0, cols, combine, 0)
        pltpu.sync_copy(res_v, out_hbm.at[pl.ds(base, cols)])

    return sc_pool


def _tc_repack(emb_t, v_blk):
    emb, vocab = emb_t.shape
    wide = 2 * emb
    hblk = v_blk // 2

    def body(a_ref, o_ref):
        t = a_ref[...].T
        o_ref[...] = jnp.concatenate([t[:hblk], t[hblk:]], axis=1)

    return pl.pallas_call(
        body,
        grid=(pl.cdiv(vocab, v_blk),),
        in_specs=[pl.BlockSpec((emb, v_blk), lambda i: (0, i))],
        out_specs=pl.BlockSpec((hblk, wide), lambda i: (i, 0)),
        out_shape=jax.ShapeDtypeStruct(
            (pl.cdiv(vocab, v_blk) * hblk, wide), jnp.float32),
    )(emb_t)


def _tc_head(sums, fc_w, fc_b2d, seq, blk):
    batch, emb = sums.shape
    out_dim = fc_w.shape[0]
    inv = 1.0 / seq

    def body(s_ref, w_ref, b_ref, o_ref):
        s = s_ref[...]
        w = w_ref[...]
        logits = lax.dot_general(
            s, w, (((1,), (1,)), ((), ())),
            preferred_element_type=jnp.float32,
        ) * inv + b_ref[...]
        m = jnp.max(logits, axis=-1, keepdims=True)
        e = jnp.exp(logits - m)
        lse = jnp.log(jnp.sum(e, axis=-1, keepdims=True)) + m
        o_ref[...] = logits - lse

    return pl.pallas_call(
        body,
        grid=(batch // blk,),
        in_specs=[
            pl.BlockSpec((blk, emb), lambda i: (i, 0)),
            pl.BlockSpec((out_dim, emb), lambda i: (0, 0)),
            pl.BlockSpec((1, out_dim), lambda i: (0, 0)),
        ],
        out_specs=pl.BlockSpec((blk, out_dim), lambda i: (i, 0)),
        out_shape=jax.ShapeDtypeStruct((batch, out_dim), jnp.float32),
    )(sums, fc_w, fc_b2d)


def kernel(x, embedding, fc_w, fc_b):
    seq, batch = x.shape
    vocab, emb = embedding.shape
    vblk = 32768
    emb_dense = _tc_repack(embedding.T, v_blk=vblk).reshape(-1, emb)
    sums = _make_sc_pool(seq, batch, emb, vblk, 14)(x, emb_dense)
    return _tc_head(sums, fc_w, fc_b.reshape(1, -1), seq, blk=512)
